# baseline (speedup 1.0000x reference)
import functools

import jax
import jax.numpy as jnp
from jax import lax
from jax.experimental import pallas as pl
from jax.experimental.pallas import tpu as pltpu
from jax.experimental.pallas import tpu_sc as plsc

_NR = 4
_B = 2048
_D = 128
_NCTX = 4096
_T = _NR * _B
_AVG_N = 50000.0

_TOK_BLK = 2048
_N_TOK_BLKS = _T // _TOK_BLK


def _k1_body(x_ref, ctx_ref, cm_ref, argm_ref, mseg_ref, cn_ref):
    pid = pl.program_id(0)

    @pl.when(pid == 0)
    def _prep():
        ctx = ctx_ref[...]
        nrm = jnp.sqrt(jnp.sum(ctx * ctx, axis=1, keepdims=True))
        cn_ref[...] = ctx / (nrm + 1e-8)
        seg = lax.dot_general(ctx, cm_ref[...], (((1,), (1,)), ((), ())),
                              preferred_element_type=jnp.float32)
        mseg_ref[...] = jnp.reshape(jnp.max(seg, axis=1), (_NCTX // _D, _D))

    half = _TOK_BLK // 4
    for h in range(4):
        xs = x_ref[0, h * half:(h + 1) * half, :]
        sim = lax.dot_general(xs, cn_ref[...], (((1,), (1,)), ((), ())),
                              preferred_element_type=jnp.float32)
        am = jnp.argmax(sim, axis=1).astype(jnp.int32)
        argm_ref[h * (half // _D):(h + 1) * (half // _D), :] = (
            jnp.reshape(am, (half // _D, _D)))


def _k3_body(x_ref, w_ref, b_ref, a0_ref, a1_ref, a2_ref, a3_ref, o_ref):
    w = w_ref[...]
    blk = o_ref.shape[0]
    acc = None
    for n, a_ref in enumerate((a0_ref, a1_ref, a2_ref, a3_ref)):
        h = jnp.dot(x_ref[n], w, preferred_element_type=jnp.float32) + b_ref[...]
        g = jax.nn.gelu(h)
        parts = []
        for r in range(blk // _D):
            a_col = jnp.transpose(a_ref[r:r + 1, :])
            parts.append(g[r * _D:(r + 1) * _D, :] * a_col)
        g = jnp.concatenate(parts, axis=0)
        acc = g if acc is None else acc + g
    o_ref[...] = acc * (1.0 / _NR)


def _make_k2(nc, ns):
    nw = nc * ns
    rows_w = _NCTX // nw
    tok_w = _T // nw
    chunk = 128
    assert tok_w == 2 * chunk
    mesh = plsc.VectorSubcoreMesh(core_axis_name="c", subcore_axis_name="s")

    @functools.partial(
        pl.kernel,
        out_type=(
            jax.ShapeDtypeStruct((_NCTX, _D), jnp.float32),
            jax.ShapeDtypeStruct((_T,), jnp.float32),
        ),
        mesh=mesh,
        compiler_params=pltpu.CompilerParams(needs_layout_passes=False),
        scratch_types=[
            pltpu.VMEM((chunk, _D), jnp.float32),
            pltpu.VMEM((chunk, _D), jnp.float32),
            pltpu.VMEM((chunk, _D), jnp.float32),
            pltpu.VMEM((chunk, _D), jnp.float32),
            pltpu.VMEM((chunk,), jnp.int32),
            pltpu.VMEM((chunk,), jnp.int32),
            pltpu.VMEM((_NCTX,), jnp.float32),
            pltpu.VMEM((chunk,), jnp.float32),
            pltpu.VMEM((_NCTX // nw, _D), jnp.float32),
            pltpu.SemaphoreType.DMA,
            pltpu.SemaphoreType.DMA,
            pltpu.SemaphoreType.DMA,
            pltpu.SemaphoreType.DMA,
            pltpu.SemaphoreType.DMA,
        ],
    )
    def k2(ctx_hbm, x_hbm, argm_hbm, mseg_hbm, newctx_hbm, act_hbm,
           buf0, buf1, x0, x1, idx0, idx1, mseg_v, act_v, cp_v,
           sg0, sg1, sx0, sx1, ss):
        wid = lax.axis_index("s") * nc + lax.axis_index("c")
        base0 = wid * tok_w
        base1 = base0 + chunk
        pltpu.sync_copy(argm_hbm.at[pl.ds(base0, chunk)], idx0)
        pltpu.sync_copy(argm_hbm.at[pl.ds(base1, chunk)], idx1)
        g0 = pltpu.async_copy(ctx_hbm.at[idx0], buf0, sg0)
        g1 = pltpu.async_copy(ctx_hbm.at[idx1], buf1, sg1)
        c0 = pltpu.async_copy(x_hbm.at[pl.ds(base0, chunk)], x0, sx0)
        c1 = pltpu.async_copy(x_hbm.at[pl.ds(base1, chunk)], x1, sx1)
        r0 = wid * rows_w
        pltpu.sync_copy(ctx_hbm.at[pl.ds(r0, rows_w)], cp_v)
        pltpu.sync_copy(cp_v, newctx_hbm.at[pl.ds(r0, rows_w)])
        pltpu.sync_copy(mseg_hbm, mseg_v)
        plsc.subcore_barrier()

        def do_act(idx, base):
            for j in range(chunk // 16):
                idx16 = idx[pl.ds(j * 16, 16)]
                m = plsc.load_gather(mseg_v, [idx16])
                act_v[pl.ds(j * 16, 16)] = 1.0 / (1.0 + jnp.exp(-m))
            pltpu.sync_copy(act_v, act_hbm.at[pl.ds(base, chunk)])

        do_act(idx0, base0)
        do_act(idx1, base1)

        def do_chunk(gd, cd, buf, xv, idx, sdone):
            gd.wait()
            cd.wait()

            @functools.partial(plsc.parallel_loop, 0, chunk, unroll=8)
            def _row(i):
                for j in range(_D // 16):
                    sl = (i, pl.ds(j * 16, 16))
                    buf[sl] = (buf[sl] * (_AVG_N - 1.0) + xv[sl]) * (1.0 / _AVG_N)

            return pltpu.async_copy(buf, newctx_hbm.at[idx], sdone)

        s0 = do_chunk(g0, c0, buf0, x0, idx0, ss)
        s1 = do_chunk(g1, c1, buf1, x1, idx1, ss)
        s0.wait()
        s1.wait()

    return k2


def kernel(x, W, b, ctx_mod, context):
    xf = jnp.reshape(x, (_T, _D))

    argm2, mseg2 = pl.pallas_call(
        _k1_body,
        grid=(_N_TOK_BLKS,),
        in_specs=[
            pl.BlockSpec((1, _TOK_BLK, _D), lambda i: (i, 0, 0)),
            pl.BlockSpec((_NCTX, _D), lambda i: (0, 0)),
            pl.BlockSpec((_NR, _D), lambda i: (0, 0)),
        ],
        out_specs=[
            pl.BlockSpec((_TOK_BLK // _D, _D), lambda i: (i, 0)),
            pl.BlockSpec((_NCTX // _D, _D), lambda i: (0, 0)),
        ],
        out_shape=[
            jax.ShapeDtypeStruct((_T // _D, _D), jnp.int32),
            jax.ShapeDtypeStruct((_NCTX // _D, _D), jnp.float32),
        ],
        scratch_shapes=[pltpu.VMEM((_NCTX, _D), jnp.float32)],
    )(jnp.reshape(xf, (_N_TOK_BLKS, _TOK_BLK, _D)), context, ctx_mod)

    argm = jnp.reshape(argm2, (_T,))
    mseg = jnp.reshape(mseg2, (_NCTX,))

    info = plsc.get_sparse_core_info()
    k2 = _make_k2(info.num_cores, info.num_subcores)
    new_context, act = k2(context, xf, argm, mseg)

    blk = 2048
    act64 = jnp.reshape(act, (_T // _D, _D))

    def _act_map(n):
        return lambda i: (n * (_B // blk) + i, 0)

    act_specs = [pl.BlockSpec((blk // _D, _D), _act_map(n)) for n in range(_NR)]
    x_out = pl.pallas_call(
        _k3_body,
        grid=(_B // blk,),
        in_specs=[
            pl.BlockSpec((_NR, blk, _D), lambda i: (0, i, 0)),
            pl.BlockSpec((_D, _D), lambda i: (0, 0)),
            pl.BlockSpec((1, _D), lambda i: (0, 0)),
        ] + act_specs,
        out_specs=pl.BlockSpec((blk, _D), lambda i: (i, 0)),
        out_shape=jax.ShapeDtypeStruct((_B, _D), jnp.float32),
    )(x, W, jnp.reshape(b, (1, _D)), act64, act64, act64, act64)

    return (x_out, new_context)


# speedup vs baseline: 1.0073x; 1.0073x over previous
import functools

import jax
import jax.numpy as jnp
from jax import lax
from jax.experimental import pallas as pl
from jax.experimental.pallas import tpu as pltpu
from jax.experimental.pallas import tpu_sc as plsc

_NR = 4
_B = 2048
_D = 128
_NCTX = 4096
_T = _NR * _B
_AVG_N = 50000.0

_TOK_BLK = 2048
_N_TOK_BLKS = _T // _TOK_BLK


def _k1_body(x_ref, ctx_ref, cm_ref, argm_ref, mseg_ref, cn_ref):
    pid = pl.program_id(0)

    @pl.when(pid == 0)
    def _prep():
        ctx = ctx_ref[...]
        nrm = jnp.sqrt(jnp.sum(ctx * ctx, axis=1, keepdims=True))
        cn_ref[...] = ctx / (nrm + 1e-8)
        seg = lax.dot_general(ctx, cm_ref[...], (((1,), (1,)), ((), ())),
                              preferred_element_type=jnp.float32)
        mseg_ref[...] = jnp.reshape(jnp.max(seg, axis=1), (_NCTX // _D, _D))

    half = _TOK_BLK // 4
    for h in range(4):
        xs = x_ref[0, h * half:(h + 1) * half, :]
        sim = lax.dot_general(xs, cn_ref[...], (((1,), (1,)), ((), ())),
                              preferred_element_type=jnp.float32)
        am = jnp.argmax(sim, axis=1).astype(jnp.int32)
        argm_ref[h * (half // _D):(h + 1) * (half // _D), :] = (
            jnp.reshape(am, (half // _D, _D)))


def _k3_body(x_ref, w_ref, b_ref, a0_ref, a1_ref, a2_ref, a3_ref, o_ref):
    w = w_ref[...]
    blk = o_ref.shape[0]
    acc = None
    for n, a_ref in enumerate((a0_ref, a1_ref, a2_ref, a3_ref)):
        h = jnp.dot(x_ref[n], w, preferred_element_type=jnp.float32) + b_ref[...]
        g = jax.nn.gelu(h)
        parts = []
        for r in range(blk // _D):
            a_col = jnp.transpose(a_ref[r:r + 1, :])
            parts.append(g[r * _D:(r + 1) * _D, :] * a_col)
        g = jnp.concatenate(parts, axis=0)
        acc = g if acc is None else acc + g
    o_ref[...] = acc * (1.0 / _NR)


def _make_k2(nc, ns):
    nw = nc * ns
    rows_w = _NCTX // nw
    tok_w = _T // nw
    chunk = 128
    assert tok_w == 2 * chunk
    mesh = plsc.VectorSubcoreMesh(core_axis_name="c", subcore_axis_name="s")

    @functools.partial(
        pl.kernel,
        out_type=(
            jax.ShapeDtypeStruct((_NCTX, _D), jnp.float32),
            jax.ShapeDtypeStruct((_T,), jnp.float32),
        ),
        mesh=mesh,
        compiler_params=pltpu.CompilerParams(needs_layout_passes=False),
        scratch_types=[
            pltpu.VMEM((chunk, _D), jnp.float32),
            pltpu.VMEM((chunk, _D), jnp.float32),
            pltpu.VMEM((chunk, _D), jnp.float32),
            pltpu.VMEM((chunk, _D), jnp.float32),
            pltpu.VMEM((chunk,), jnp.int32),
            pltpu.VMEM((chunk,), jnp.int32),
            pltpu.VMEM((_NCTX,), jnp.float32),
            pltpu.VMEM((chunk,), jnp.float32),
            pltpu.VMEM((_NCTX // nw, _D), jnp.float32),
            pltpu.SemaphoreType.DMA,
            pltpu.SemaphoreType.DMA,
            pltpu.SemaphoreType.DMA,
            pltpu.SemaphoreType.DMA,
            pltpu.SemaphoreType.DMA,
        ],
    )
    def k2(ctx_hbm, x_hbm, argm_hbm, mseg_hbm, newctx_hbm, act_hbm,
           buf0, buf1, x0, x1, idx0, idx1, mseg_v, act_v, cp_v,
           sg0, sg1, sx0, sx1, ss):
        wid = lax.axis_index("s") * nc + lax.axis_index("c")
        base0 = wid * tok_w
        base1 = base0 + chunk
        pltpu.sync_copy(argm_hbm.at[pl.ds(base0, chunk)], idx0)
        pltpu.sync_copy(argm_hbm.at[pl.ds(base1, chunk)], idx1)
        g0 = pltpu.async_copy(ctx_hbm.at[idx0], buf0, sg0)
        g1 = pltpu.async_copy(ctx_hbm.at[idx1], buf1, sg1)
        c0 = pltpu.async_copy(x_hbm.at[pl.ds(base0, chunk)], x0, sx0)
        c1 = pltpu.async_copy(x_hbm.at[pl.ds(base1, chunk)], x1, sx1)
        r0 = wid * rows_w
        pltpu.sync_copy(ctx_hbm.at[pl.ds(r0, rows_w)], cp_v)
        pltpu.sync_copy(cp_v, newctx_hbm.at[pl.ds(r0, rows_w)])
        pltpu.sync_copy(mseg_hbm, mseg_v)
        plsc.subcore_barrier()

        def do_act(idx, base):
            for j in range(chunk // 16):
                idx16 = idx[pl.ds(j * 16, 16)]
                m = plsc.load_gather(mseg_v, [idx16])
                act_v[pl.ds(j * 16, 16)] = 1.0 / (1.0 + jnp.exp(-m))
            pltpu.sync_copy(act_v, act_hbm.at[pl.ds(base, chunk)])

        do_act(idx0, base0)
        do_act(idx1, base1)

        def do_chunk(gd, cd, buf, xv, idx, sdone):
            gd.wait()
            cd.wait()

            @functools.partial(plsc.parallel_loop, 0, chunk, unroll=8)
            def _row(i):
                for j in range(_D // 16):
                    sl = (i, pl.ds(j * 16, 16))
                    buf[sl] = (buf[sl] * (_AVG_N - 1.0) + xv[sl]) * (1.0 / _AVG_N)

            return pltpu.async_copy(buf, newctx_hbm.at[idx], sdone)

        s0 = do_chunk(g0, c0, buf0, x0, idx0, ss)
        s1 = do_chunk(g1, c1, buf1, x1, idx1, ss)
        s0.wait()
        s1.wait()

    return k2


def kernel(x, W, b, ctx_mod, context):
    xf = jnp.reshape(x, (_T, _D))

    argm2, mseg2 = pl.pallas_call(
        _k1_body,
        grid=(_N_TOK_BLKS,),
        in_specs=[
            pl.BlockSpec((1, _TOK_BLK, _D), lambda i: (i, 0, 0)),
            pl.BlockSpec((_NCTX, _D), lambda i: (0, 0)),
            pl.BlockSpec((_NR, _D), lambda i: (0, 0)),
        ],
        out_specs=[
            pl.BlockSpec((_TOK_BLK // _D, _D), lambda i: (i, 0)),
            pl.BlockSpec((_NCTX // _D, _D), lambda i: (0, 0)),
        ],
        out_shape=[
            jax.ShapeDtypeStruct((_T // _D, _D), jnp.int32),
            jax.ShapeDtypeStruct((_NCTX // _D, _D), jnp.float32),
        ],
        scratch_shapes=[pltpu.VMEM((_NCTX, _D), jnp.float32)],
    )(jnp.reshape(xf, (_N_TOK_BLKS, _TOK_BLK, _D)), context, ctx_mod)

    argm = jnp.reshape(argm2, (_T,))
    mseg = jnp.reshape(mseg2, (_NCTX,))

    info = plsc.get_sparse_core_info()
    k2 = _make_k2(info.num_cores, info.num_subcores)
    new_context, act = k2(context, xf, argm, mseg)

    blk = 1024
    act64 = jnp.reshape(act, (_T // _D, _D))

    def _act_map(n):
        return lambda i: (n * (_B // blk) + i, 0)

    act_specs = [pl.BlockSpec((blk // _D, _D), _act_map(n)) for n in range(_NR)]
    x_out = pl.pallas_call(
        _k3_body,
        grid=(_B // blk,),
        in_specs=[
            pl.BlockSpec((_NR, blk, _D), lambda i: (0, i, 0)),
            pl.BlockSpec((_D, _D), lambda i: (0, 0)),
            pl.BlockSpec((1, _D), lambda i: (0, 0)),
        ] + act_specs,
        out_specs=pl.BlockSpec((blk, _D), lambda i: (i, 0)),
        out_shape=jax.ShapeDtypeStruct((_B, _D), jnp.float32),
    )(x, W, jnp.reshape(b, (1, _D)), act64, act64, act64, act64)

    return (x_out, new_context)


# speedup vs baseline: 1.0126x; 1.0052x over previous
import functools

import jax
import jax.numpy as jnp
from jax import lax
from jax.experimental import pallas as pl
from jax.experimental.pallas import tpu as pltpu
from jax.experimental.pallas import tpu_sc as plsc

_NR = 4
_B = 2048
_D = 128
_NCTX = 4096
_T = _NR * _B
_AVG_N = 50000.0

_TOK_BLK = 2048
_N_TOK_BLKS = _T // _TOK_BLK


def _k1_body(x_ref, ctx_ref, cm_ref, argm_ref, mseg_ref, cn_ref):
    pid = pl.program_id(0)

    @pl.when(pid == 0)
    def _prep():
        ctx = ctx_ref[...]
        nrm = jnp.sqrt(jnp.sum(ctx * ctx, axis=1, keepdims=True))
        cn_ref[...] = ctx / (nrm + 1e-8)
        seg = lax.dot_general(ctx, cm_ref[...], (((1,), (1,)), ((), ())),
                              preferred_element_type=jnp.float32)
        mseg_ref[...] = jnp.reshape(jnp.max(seg, axis=1), (_NCTX // _D, _D))

    half = _TOK_BLK // 8
    for h in range(8):
        xs = x_ref[0, h * half:(h + 1) * half, :]
        sim = lax.dot_general(xs, cn_ref[...], (((1,), (1,)), ((), ())),
                              preferred_element_type=jnp.float32)
        am = jnp.argmax(sim, axis=1).astype(jnp.int32)
        argm_ref[h * (half // _D):(h + 1) * (half // _D), :] = (
            jnp.reshape(am, (half // _D, _D)))


def _k3_body(x_ref, w_ref, b_ref, a0_ref, a1_ref, a2_ref, a3_ref, o_ref):
    w = w_ref[...]
    blk = o_ref.shape[0]
    acc = None
    for n, a_ref in enumerate((a0_ref, a1_ref, a2_ref, a3_ref)):
        h = jnp.dot(x_ref[n], w, preferred_element_type=jnp.float32) + b_ref[...]
        g = jax.nn.gelu(h)
        parts = []
        for r in range(blk // _D):
            a_col = jnp.transpose(a_ref[r:r + 1, :])
            parts.append(g[r * _D:(r + 1) * _D, :] * a_col)
        g = jnp.concatenate(parts, axis=0)
        acc = g if acc is None else acc + g
    o_ref[...] = acc * (1.0 / _NR)


def _make_k2(nc, ns):
    nw = nc * ns
    rows_w = _NCTX // nw
    tok_w = _T // nw
    chunk = 128
    assert tok_w == 2 * chunk
    mesh = plsc.VectorSubcoreMesh(core_axis_name="c", subcore_axis_name="s")

    @functools.partial(
        pl.kernel,
        out_type=(
            jax.ShapeDtypeStruct((_NCTX, _D), jnp.float32),
            jax.ShapeDtypeStruct((_T,), jnp.float32),
        ),
        mesh=mesh,
        compiler_params=pltpu.CompilerParams(needs_layout_passes=False),
        scratch_types=[
            pltpu.VMEM((chunk, _D), jnp.float32),
            pltpu.VMEM((chunk, _D), jnp.float32),
            pltpu.VMEM((chunk, _D), jnp.float32),
            pltpu.VMEM((chunk, _D), jnp.float32),
            pltpu.VMEM((chunk,), jnp.int32),
            pltpu.VMEM((chunk,), jnp.int32),
            pltpu.VMEM((_NCTX,), jnp.float32),
            pltpu.VMEM((chunk,), jnp.float32),
            pltpu.VMEM((_NCTX // nw, _D), jnp.float32),
            pltpu.SemaphoreType.DMA,
            pltpu.SemaphoreType.DMA,
            pltpu.SemaphoreType.DMA,
            pltpu.SemaphoreType.DMA,
            pltpu.SemaphoreType.DMA,
        ],
    )
    def k2(ctx_hbm, x_hbm, argm_hbm, mseg_hbm, newctx_hbm, act_hbm,
           buf0, buf1, x0, x1, idx0, idx1, mseg_v, act_v, cp_v,
           sg0, sg1, sx0, sx1, ss):
        wid = lax.axis_index("s") * nc + lax.axis_index("c")
        base0 = wid * tok_w
        base1 = base0 + chunk
        pltpu.sync_copy(argm_hbm.at[pl.ds(base0, chunk)], idx0)
        pltpu.sync_copy(argm_hbm.at[pl.ds(base1, chunk)], idx1)
        g0 = pltpu.async_copy(ctx_hbm.at[idx0], buf0, sg0)
        g1 = pltpu.async_copy(ctx_hbm.at[idx1], buf1, sg1)
        c0 = pltpu.async_copy(x_hbm.at[pl.ds(base0, chunk)], x0, sx0)
        c1 = pltpu.async_copy(x_hbm.at[pl.ds(base1, chunk)], x1, sx1)
        r0 = wid * rows_w
        pltpu.sync_copy(ctx_hbm.at[pl.ds(r0, rows_w)], cp_v)
        pltpu.sync_copy(cp_v, newctx_hbm.at[pl.ds(r0, rows_w)])
        pltpu.sync_copy(mseg_hbm, mseg_v)
        plsc.subcore_barrier()

        def do_act(idx, base):
            for j in range(chunk // 16):
                idx16 = idx[pl.ds(j * 16, 16)]
                m = plsc.load_gather(mseg_v, [idx16])
                act_v[pl.ds(j * 16, 16)] = 1.0 / (1.0 + jnp.exp(-m))
            pltpu.sync_copy(act_v, act_hbm.at[pl.ds(base, chunk)])

        do_act(idx0, base0)
        do_act(idx1, base1)

        def do_chunk(gd, cd, buf, xv, idx, sdone):
            gd.wait()
            cd.wait()

            @functools.partial(plsc.parallel_loop, 0, chunk, unroll=8)
            def _row(i):
                for j in range(_D // 16):
                    sl = (i, pl.ds(j * 16, 16))
                    buf[sl] = (buf[sl] * (_AVG_N - 1.0) + xv[sl]) * (1.0 / _AVG_N)

            return pltpu.async_copy(buf, newctx_hbm.at[idx], sdone)

        s0 = do_chunk(g0, c0, buf0, x0, idx0, ss)
        s1 = do_chunk(g1, c1, buf1, x1, idx1, ss)
        s0.wait()
        s1.wait()

    return k2


def kernel(x, W, b, ctx_mod, context):
    xf = jnp.reshape(x, (_T, _D))

    argm2, mseg2 = pl.pallas_call(
        _k1_body,
        grid=(_N_TOK_BLKS,),
        in_specs=[
            pl.BlockSpec((1, _TOK_BLK, _D), lambda i: (i, 0, 0)),
            pl.BlockSpec((_NCTX, _D), lambda i: (0, 0)),
            pl.BlockSpec((_NR, _D), lambda i: (0, 0)),
        ],
        out_specs=[
            pl.BlockSpec((_TOK_BLK // _D, _D), lambda i: (i, 0)),
            pl.BlockSpec((_NCTX // _D, _D), lambda i: (0, 0)),
        ],
        out_shape=[
            jax.ShapeDtypeStruct((_T // _D, _D), jnp.int32),
            jax.ShapeDtypeStruct((_NCTX // _D, _D), jnp.float32),
        ],
        scratch_shapes=[pltpu.VMEM((_NCTX, _D), jnp.float32)],
    )(jnp.reshape(xf, (_N_TOK_BLKS, _TOK_BLK, _D)), context, ctx_mod)

    argm = jnp.reshape(argm2, (_T,))
    mseg = jnp.reshape(mseg2, (_NCTX,))

    info = plsc.get_sparse_core_info()
    k2 = _make_k2(info.num_cores, info.num_subcores)
    new_context, act = k2(context, xf, argm, mseg)

    blk = 1024
    act64 = jnp.reshape(act, (_T // _D, _D))

    def _act_map(n):
        return lambda i: (n * (_B // blk) + i, 0)

    act_specs = [pl.BlockSpec((blk // _D, _D), _act_map(n)) for n in range(_NR)]
    x_out = pl.pallas_call(
        _k3_body,
        grid=(_B // blk,),
        in_specs=[
            pl.BlockSpec((_NR, blk, _D), lambda i: (0, i, 0)),
            pl.BlockSpec((_D, _D), lambda i: (0, 0)),
            pl.BlockSpec((1, _D), lambda i: (0, 0)),
        ] + act_specs,
        out_specs=pl.BlockSpec((blk, _D), lambda i: (i, 0)),
        out_shape=jax.ShapeDtypeStruct((_B, _D), jnp.float32),
    )(x, W, jnp.reshape(b, (1, _D)), act64, act64, act64, act64)

    return (x_out, new_context)
